# baseline (speedup 1.0000x reference)
import functools
import math

import jax
import jax.numpy as jnp
from jax import lax
from jax.experimental import pallas as pl
from jax.experimental.pallas import tpu as pltpu
from jax.experimental.pallas import tpu_sc as plsc

EMBED_DIM = 32
SCALE = math.sqrt(float(EMBED_DIM))
NUM_CORES = 2
NUM_SUBCORES = 16
NUM_WORKERS = NUM_CORES * NUM_SUBCORES
LANES = 16


def _make_sc_lookup(n_rows: int, n_cols: int, chunk_rows: int):
    assert n_rows % NUM_WORKERS == 0
    rows_per_w = n_rows // NUM_WORKERS
    assert rows_per_w % chunk_rows == 0
    n_chunks = rows_per_w // chunk_rows
    assert n_chunks % 2 == 0
    chunk = chunk_rows * n_cols
    assert chunk % LANES == 0
    n_vecs = chunk // LANES

    mesh = plsc.VectorSubcoreMesh(
        core_axis_name="c", subcore_axis_name="s",
        num_cores=NUM_CORES, num_subcores=NUM_SUBCORES)

    @functools.partial(
        pl.kernel,
        out_type=jax.ShapeDtypeStruct((n_rows, n_cols, EMBED_DIM),
                                      jnp.float32),
        mesh=mesh,
        scratch_types=[
            pltpu.VMEM((rows_per_w, n_cols), jnp.int32),
            pltpu.VMEM((chunk,), jnp.int32),
            pltpu.VMEM((chunk,), jnp.int32),
            pltpu.VMEM((chunk, EMBED_DIM), jnp.float32),
            pltpu.VMEM((chunk, EMBED_DIM), jnp.float32),
            pltpu.VMEM((chunk_rows, n_cols, EMBED_DIM), jnp.float32),
            pltpu.SemaphoreType.DMA,
            pltpu.SemaphoreType.DMA,
        ],
        compiler_params=pltpu.CompilerParams(use_tc_tiling_on_sc=False,
                                             needs_layout_passes=False),
    )
    def sc_lookup(table_hbm, idx_hbm, out_hbm, idx_v, fidx0, fidx1, flat0,
                  flat1, struct, sem0, sem1):
        wid = lax.axis_index("s") * NUM_CORES + lax.axis_index("c")
        row0 = wid * rows_per_w

        pltpu.sync_copy(idx_hbm.at[pl.ds(row0, rows_per_w)], idx_v)

        lane = lax.broadcasted_iota(jnp.int32, (LANES,), 0)

        def prep(g, fidx, sem, flat):
            r_init = jnp.full((LANES,), g * chunk_rows, jnp.int32)

            def flatten_body(j, carry):
                r_vec, c_vec = carry
                fidx[pl.ds(j * LANES, LANES)] = plsc.load_gather(
                    idx_v, [r_vec, c_vec])
                c_next = c_vec + LANES
                wrap = c_next >= n_cols
                c_next = jnp.where(wrap, c_next - n_cols, c_next)
                r_next = jnp.where(wrap, r_vec + 1, r_vec)
                return r_next, c_next

            lax.fori_loop(0, n_vecs, flatten_body, (r_init, lane), unroll=5)
            pltpu.async_copy(table_hbm.at[fidx], flat, sem)

        def finish(g, fidx, sem, flat):
            pltpu.make_async_copy(table_hbm.at[fidx], flat, sem).wait()

            def scale_row(r, carry):
                base = r * n_cols

                def scale_col(c, carry2):
                    for h in range(EMBED_DIM // LANES):
                        sl = pl.ds(h * LANES, LANES)
                        struct[r, c, sl] = flat[base + c, sl] * SCALE
                    return carry2

                lax.fori_loop(0, n_cols, scale_col, 0, unroll=5)
                return carry

            lax.fori_loop(0, chunk_rows, scale_row, 0)
            pltpu.sync_copy(
                struct,
                out_hbm.at[pl.ds(row0 + g * chunk_rows, chunk_rows)])

        prep(0, fidx0, sem0, flat0)

        def pair_body(p, carry):
            g0 = 2 * p
            prep(g0 + 1, fidx1, sem1, flat1)
            finish(g0, fidx0, sem0, flat0)

            @pl.when(g0 + 2 < n_chunks)
            def _():
                prep(g0 + 2, fidx0, sem0, flat0)

            finish(g0 + 1, fidx1, sem1, flat1)
            return carry

        lax.fori_loop(0, n_chunks // 2, pair_body, 0)

    return sc_lookup


def kernel(input, table):
    n_rows, n_cols = input.shape
    idx = input.astype(jnp.int32)
    return _make_sc_lookup(n_rows, n_cols, chunk_rows=16)(table, idx)


# speedup vs baseline: 1.0110x; 1.0110x over previous
import functools
import math

import jax
import jax.numpy as jnp
from jax import lax
from jax.experimental import pallas as pl
from jax.experimental.pallas import tpu as pltpu
from jax.experimental.pallas import tpu_sc as plsc

EMBED_DIM = 32
SCALE = math.sqrt(float(EMBED_DIM))
NUM_CORES = 2
NUM_SUBCORES = 16
NUM_WORKERS = NUM_CORES * NUM_SUBCORES
LANES = 16


def _make_sc_lookup(n_rows: int, n_cols: int, chunk_rows: int):
    assert n_rows % NUM_WORKERS == 0
    rows_per_w = n_rows // NUM_WORKERS
    assert rows_per_w % chunk_rows == 0
    n_chunks = rows_per_w // chunk_rows
    assert n_chunks % 2 == 0
    chunk = chunk_rows * n_cols
    b_per_w = rows_per_w * n_cols

    mesh = plsc.VectorSubcoreMesh(
        core_axis_name="c", subcore_axis_name="s",
        num_cores=NUM_CORES, num_subcores=NUM_SUBCORES)

    @functools.partial(
        pl.kernel,
        out_type=jax.ShapeDtypeStruct((n_rows, n_cols, EMBED_DIM),
                                      jnp.float32),
        mesh=mesh,
        scratch_types=[
            pltpu.VMEM((b_per_w,), jnp.int32),
            pltpu.VMEM((chunk, EMBED_DIM), jnp.float32),
            pltpu.VMEM((chunk, EMBED_DIM), jnp.float32),
            pltpu.VMEM((chunk_rows, n_cols, EMBED_DIM), jnp.float32),
            pltpu.SemaphoreType.DMA,
            pltpu.SemaphoreType.DMA,
        ],
        compiler_params=pltpu.CompilerParams(use_tc_tiling_on_sc=False,
                                             needs_layout_passes=False),
    )
    def sc_lookup(table_hbm, idx_hbm, out_hbm, idx_v, flat0, flat1, struct,
                  sem0, sem1):
        wid = lax.axis_index("s") * NUM_CORES + lax.axis_index("c")
        row0 = wid * rows_per_w

        pltpu.sync_copy(idx_hbm.at[pl.ds(wid * b_per_w, b_per_w)], idx_v)

        def offsets(g):
            return idx_v.at[pl.ds(g * chunk, chunk)]

        def prep(g, sem, flat):
            pltpu.async_copy(table_hbm.at[offsets(g)], flat, sem)

        def finish(g, sem, flat):
            pltpu.make_async_copy(table_hbm.at[offsets(g)], flat, sem).wait()

            def scale_row(r, carry):
                base = r * n_cols

                def scale_col(c, carry2):
                    for h in range(EMBED_DIM // LANES):
                        sl = pl.ds(h * LANES, LANES)
                        struct[r, c, sl] = flat[base + c, sl] * SCALE
                    return carry2

                lax.fori_loop(0, n_cols, scale_col, 0, unroll=5)
                return carry

            lax.fori_loop(0, chunk_rows, scale_row, 0)
            pltpu.sync_copy(
                struct,
                out_hbm.at[pl.ds(row0 + g * chunk_rows, chunk_rows)])

        prep(0, sem0, flat0)

        def pair_body(p, carry):
            g0 = 2 * p
            prep(g0 + 1, sem1, flat1)
            finish(g0, sem0, flat0)

            @pl.when(g0 + 2 < n_chunks)
            def _():
                prep(g0 + 2, sem0, flat0)

            finish(g0 + 1, sem1, flat1)
            return carry

        lax.fori_loop(0, n_chunks // 2, pair_body, 0)

    return sc_lookup


def kernel(input, table):
    n_rows, n_cols = input.shape
    idx = input.reshape(-1).astype(jnp.int32)
    return _make_sc_lookup(n_rows, n_cols, chunk_rows=16)(table, idx)


# speedup vs baseline: 1.0433x; 1.0320x over previous
import functools
import math

import jax
import jax.numpy as jnp
from jax import lax
from jax.experimental import pallas as pl
from jax.experimental.pallas import tpu as pltpu
from jax.experimental.pallas import tpu_sc as plsc

EMBED_DIM = 32
SCALE = math.sqrt(float(EMBED_DIM))
NUM_CORES = 2
NUM_SUBCORES = 16
NUM_WORKERS = NUM_CORES * NUM_SUBCORES
LANES = 16


def _make_sc_lookup(n_rows: int, n_cols: int):
    assert n_rows % NUM_WORKERS == 0
    rows_per_w = n_rows // NUM_WORKERS
    assert n_cols % 2 == 0

    mesh = plsc.VectorSubcoreMesh(
        core_axis_name="c", subcore_axis_name="s",
        num_cores=NUM_CORES, num_subcores=NUM_SUBCORES)

    @functools.partial(
        pl.kernel,
        out_type=jax.ShapeDtypeStruct((n_cols, EMBED_DIM, n_rows),
                                      jnp.float32),
        mesh=mesh,
        scratch_types=[
            pltpu.VMEM((n_cols, rows_per_w), jnp.int32),
            pltpu.VMEM((rows_per_w, EMBED_DIM), jnp.float32),
            pltpu.VMEM((rows_per_w, EMBED_DIM), jnp.float32),
            pltpu.VMEM((EMBED_DIM, rows_per_w), jnp.float32),
            pltpu.VMEM((EMBED_DIM, rows_per_w), jnp.float32),
            pltpu.SemaphoreType.DMA,
            pltpu.SemaphoreType.DMA,
            pltpu.SemaphoreType.DMA,
            pltpu.SemaphoreType.DMA,
        ],
        compiler_params=pltpu.CompilerParams(use_tc_tiling_on_sc=False,
                                             needs_layout_passes=False),
    )
    def sc_lookup(table_hbm, idxt_hbm, outt_hbm, idx_v, rows0, rows1, tb0,
                  tb1, gs0, gs1, os0, os1):
        wid = lax.axis_index("s") * NUM_CORES + lax.axis_index("c")
        col0 = wid * rows_per_w

        pltpu.sync_copy(idxt_hbm.at[:, pl.ds(col0, rows_per_w)], idx_v)

        lanes = [lax.broadcasted_iota(jnp.int32, (LANES,), 0) + h * LANES
                 for h in range(EMBED_DIM // LANES)]

        def out_slice(c):
            return outt_hbm.at[c, :, pl.ds(col0, rows_per_w)]

        def start_gather(c, rows, sem):
            pltpu.async_copy(table_hbm.at[idx_v.at[c]], rows, sem)

        def finish(c, rows, tb, gsem, osem, p):
            pltpu.make_async_copy(table_hbm.at[idx_v.at[c]], rows,
                                  gsem).wait()

            @pl.when(p > 0)
            def _():
                pltpu.make_async_copy(tb, out_slice(c - 2), osem).wait()

            def scale_t(i, carry):
                icol = jnp.full((LANES,), 0, jnp.int32) + i
                for h in range(EMBED_DIM // LANES):
                    v = rows[i, pl.ds(h * LANES, LANES)] * SCALE
                    plsc.store_scatter(tb, [lanes[h], icol], v)
                return carry

            lax.fori_loop(0, rows_per_w, scale_t, 0, unroll=4)
            pltpu.async_copy(tb, out_slice(c), osem)

        start_gather(0, rows0, gs0)

        def pair_body(p, carry):
            c0 = 2 * p
            start_gather(c0 + 1, rows1, gs1)
            finish(c0, rows0, tb0, gs0, os0, p)

            @pl.when(c0 + 2 < n_cols)
            def _():
                start_gather(c0 + 2, rows0, gs0)

            finish(c0 + 1, rows1, tb1, gs1, os1, p)
            return carry

        lax.fori_loop(0, n_cols // 2, pair_body, 0)
        pltpu.make_async_copy(tb0, out_slice(n_cols - 2), os0).wait()
        pltpu.make_async_copy(tb1, out_slice(n_cols - 1), os1).wait()

    return sc_lookup


def kernel(input, table):
    n_rows, n_cols = input.shape
    idxt = input.T.astype(jnp.int32)
    outt = _make_sc_lookup(n_rows, n_cols)(table, idxt)
    return jnp.transpose(outt, (2, 0, 1))


# speedup vs baseline: 1.4177x; 1.3589x over previous
import functools
import math

import jax
import jax.numpy as jnp
from jax import lax
from jax.experimental import pallas as pl
from jax.experimental.pallas import tpu as pltpu
from jax.experimental.pallas import tpu_sc as plsc

EMBED_DIM = 32
SCALE = math.sqrt(float(EMBED_DIM))
NUM_CORES = 2
NUM_SUBCORES = 16
NUM_WORKERS = NUM_CORES * NUM_SUBCORES
LANES = 16


def _make_sc_lookup(n_rows: int, n_cols: int):
    assert n_rows % NUM_WORKERS == 0
    rows_per_w = n_rows // NUM_WORKERS
    assert n_cols % 2 == 0

    mesh = plsc.VectorSubcoreMesh(
        core_axis_name="c", subcore_axis_name="s",
        num_cores=NUM_CORES, num_subcores=NUM_SUBCORES)

    @functools.partial(
        pl.kernel,
        out_type=jax.ShapeDtypeStruct((n_cols, EMBED_DIM, n_rows),
                                      jnp.float32),
        mesh=mesh,
        scratch_types=[
            pltpu.VMEM((n_cols, rows_per_w), jnp.int32),
            pltpu.VMEM((rows_per_w, EMBED_DIM), jnp.float32),
            pltpu.VMEM((rows_per_w, EMBED_DIM), jnp.float32),
            pltpu.VMEM((EMBED_DIM, rows_per_w + 3), jnp.float32),
            pltpu.VMEM((EMBED_DIM, rows_per_w + 3), jnp.float32),
            pltpu.SemaphoreType.DMA,
            pltpu.SemaphoreType.DMA,
            pltpu.SemaphoreType.DMA,
            pltpu.SemaphoreType.DMA,
        ],
        compiler_params=pltpu.CompilerParams(use_tc_tiling_on_sc=False,
                                             needs_layout_passes=False),
    )
    def sc_lookup(table_hbm, idxt_hbm, outt_hbm, idx_v, rows0, rows1, tb0,
                  tb1, gs0, gs1, os0, os1):
        wid = lax.axis_index("s") * NUM_CORES + lax.axis_index("c")
        col0 = wid * rows_per_w

        pltpu.sync_copy(idxt_hbm.at[:, pl.ds(col0, rows_per_w)], idx_v)

        lanes = [lax.broadcasted_iota(jnp.int32, (LANES,), 0) + h * LANES
                 for h in range(EMBED_DIM // LANES)]

        def out_slice(c):
            return outt_hbm.at[c, :, pl.ds(col0, rows_per_w)]

        def start_gather(c, rows, sem):
            pltpu.async_copy(table_hbm.at[idx_v.at[c]], rows, sem)

        def finish(c, rows, tb, gsem, osem, p):
            pltpu.make_async_copy(table_hbm.at[idx_v.at[c]], rows,
                                  gsem).wait()

            @pl.when(p > 0)
            def _():
                pltpu.make_async_copy(tb.at[:, pl.ds(0, rows_per_w)],
                                      out_slice(c - 2), osem).wait()

            def scale_t(i, carry):
                icol = jnp.full((LANES,), 0, jnp.int32) + i
                for h in range(EMBED_DIM // LANES):
                    v = rows[i, pl.ds(h * LANES, LANES)] * SCALE
                    plsc.store_scatter(tb, [lanes[h], icol], v)
                return carry

            lax.fori_loop(0, rows_per_w, scale_t, 0, unroll=4)
            pltpu.async_copy(tb.at[:, pl.ds(0, rows_per_w)], out_slice(c),
                             osem)

        start_gather(0, rows0, gs0)

        def pair_body(p, carry):
            c0 = 2 * p
            start_gather(c0 + 1, rows1, gs1)
            finish(c0, rows0, tb0, gs0, os0, p)

            @pl.when(c0 + 2 < n_cols)
            def _():
                start_gather(c0 + 2, rows0, gs0)

            finish(c0 + 1, rows1, tb1, gs1, os1, p)
            return carry

        lax.fori_loop(0, n_cols // 2, pair_body, 0)
        pltpu.make_async_copy(tb0.at[:, pl.ds(0, rows_per_w)],
                              out_slice(n_cols - 2), os0).wait()
        pltpu.make_async_copy(tb1.at[:, pl.ds(0, rows_per_w)],
                              out_slice(n_cols - 1), os1).wait()

    return sc_lookup


def kernel(input, table):
    n_rows, n_cols = input.shape
    idxt = input.T.astype(jnp.int32)
    outt = _make_sc_lookup(n_rows, n_cols)(table, idxt)
    return jnp.transpose(outt, (2, 0, 1))


# speedup vs baseline: 1.4227x; 1.0035x over previous
import functools
import math

import jax
import jax.numpy as jnp
from jax import lax
from jax.experimental import pallas as pl
from jax.experimental.pallas import tpu as pltpu
from jax.experimental.pallas import tpu_sc as plsc

EMBED_DIM = 32
SCALE = math.sqrt(float(EMBED_DIM))
NUM_CORES = 2
NUM_SUBCORES = 16
NUM_WORKERS = NUM_CORES * NUM_SUBCORES
LANES = 16


def _make_sc_lookup(n_rows: int, n_cols: int):
    assert n_rows % NUM_WORKERS == 0
    rows_per_w = n_rows // NUM_WORKERS
    assert n_cols % 2 == 0

    mesh = plsc.VectorSubcoreMesh(
        core_axis_name="c", subcore_axis_name="s",
        num_cores=NUM_CORES, num_subcores=NUM_SUBCORES)

    @functools.partial(
        pl.kernel,
        out_type=jax.ShapeDtypeStruct((n_cols, EMBED_DIM, n_rows),
                                      jnp.float32),
        mesh=mesh,
        scratch_types=[
            pltpu.VMEM((n_cols, rows_per_w), jnp.int32),
            pltpu.VMEM((rows_per_w, EMBED_DIM), jnp.float32),
            pltpu.VMEM((rows_per_w, EMBED_DIM), jnp.float32),
            pltpu.VMEM((EMBED_DIM, rows_per_w + 3), jnp.float32),
            pltpu.VMEM((EMBED_DIM, rows_per_w + 3), jnp.float32),
            pltpu.SemaphoreType.DMA,
            pltpu.SemaphoreType.DMA,
            pltpu.SemaphoreType.DMA,
            pltpu.SemaphoreType.DMA,
        ],
        compiler_params=pltpu.CompilerParams(use_tc_tiling_on_sc=False,
                                             needs_layout_passes=False),
    )
    def sc_lookup(table_hbm, idxt_hbm, outt_hbm, idx_v, rows0, rows1, tb0,
                  tb1, gs0, gs1, os0, os1):
        wid = lax.axis_index("s") * NUM_CORES + lax.axis_index("c")
        col0 = wid * rows_per_w

        pltpu.sync_copy(idxt_hbm.at[:, pl.ds(col0, rows_per_w)], idx_v)

        lanes = [lax.broadcasted_iota(jnp.int32, (LANES,), 0) + h * LANES
                 for h in range(EMBED_DIM // LANES)]

        def out_slice(c):
            return outt_hbm.at[c, :, pl.ds(col0, rows_per_w)]

        def start_gather(c, rows, sem):
            pltpu.async_copy(table_hbm.at[idx_v.at[c]], rows, sem)

        def finish(c, rows, tb, gsem, osem, p):
            pltpu.make_async_copy(table_hbm.at[idx_v.at[c]], rows,
                                  gsem).wait()

            @pl.when(p > 0)
            def _():
                pltpu.make_async_copy(tb.at[:, pl.ds(0, rows_per_w)],
                                      out_slice(c - 2), osem).wait()

            def scale_t(i, carry):
                icol = jnp.full((LANES,), 0, jnp.int32) + i
                for h in range(EMBED_DIM // LANES):
                    v = rows[i, pl.ds(h * LANES, LANES)] * SCALE
                    plsc.store_scatter(tb, [lanes[h], icol], v)
                return carry

            lax.fori_loop(0, rows_per_w, scale_t, 0, unroll=16)
            pltpu.async_copy(tb.at[:, pl.ds(0, rows_per_w)], out_slice(c),
                             osem)

        start_gather(0, rows0, gs0)

        def pair_body(p, carry):
            c0 = 2 * p
            start_gather(c0 + 1, rows1, gs1)
            finish(c0, rows0, tb0, gs0, os0, p)

            @pl.when(c0 + 2 < n_cols)
            def _():
                start_gather(c0 + 2, rows0, gs0)

            finish(c0 + 1, rows1, tb1, gs1, os1, p)
            return carry

        lax.fori_loop(0, n_cols // 2, pair_body, 0)
        pltpu.make_async_copy(tb0.at[:, pl.ds(0, rows_per_w)],
                              out_slice(n_cols - 2), os0).wait()
        pltpu.make_async_copy(tb1.at[:, pl.ds(0, rows_per_w)],
                              out_slice(n_cols - 1), os1).wait()

    return sc_lookup


def kernel(input, table):
    n_rows, n_cols = input.shape
    idxt = input.T.astype(jnp.int32)
    outt = _make_sc_lookup(n_rows, n_cols)(table, idxt)
    return jnp.transpose(outt, (2, 0, 1))


# speedup vs baseline: 1.4348x; 1.0085x over previous
import functools
import math

import jax
import jax.numpy as jnp
from jax import lax
from jax.experimental import pallas as pl
from jax.experimental.pallas import tpu as pltpu
from jax.experimental.pallas import tpu_sc as plsc

EMBED_DIM = 32
SCALE = math.sqrt(float(EMBED_DIM))
NUM_CORES = 2
NUM_SUBCORES = 16
NUM_WORKERS = NUM_CORES * NUM_SUBCORES
LANES = 16
TABLE_PITCH = 128


def _make_sc_lookup(n_rows: int, n_cols: int):
    assert n_rows % NUM_WORKERS == 0
    rows_per_w = n_rows // NUM_WORKERS
    half = rows_per_w // 2
    n_chunks = 2 * n_cols
    pitch = half + 3

    mesh = plsc.VectorSubcoreMesh(
        core_axis_name="c", subcore_axis_name="s",
        num_cores=NUM_CORES, num_subcores=NUM_SUBCORES)

    @functools.partial(
        pl.kernel,
        out_type=jax.ShapeDtypeStruct((n_cols, EMBED_DIM, n_rows),
                                      jnp.float32),
        mesh=mesh,
        scratch_types=[
            pltpu.VMEM((n_cols, rows_per_w), jnp.int32),
            pltpu.VMEM((half, TABLE_PITCH), jnp.float32),
            pltpu.VMEM((half, TABLE_PITCH), jnp.float32),
            pltpu.VMEM((EMBED_DIM, pitch), jnp.float32),
            pltpu.VMEM((EMBED_DIM, pitch), jnp.float32),
            pltpu.SemaphoreType.DMA,
            pltpu.SemaphoreType.DMA,
            pltpu.SemaphoreType.DMA,
            pltpu.SemaphoreType.DMA,
        ],
        compiler_params=pltpu.CompilerParams(use_tc_tiling_on_sc=False,
                                             needs_layout_passes=False),
    )
    def sc_lookup(table_hbm, idxt_hbm, outt_hbm, idx_v, rows0, rows1, tb0,
                  tb1, gs0, gs1, os0, os1):
        wid = lax.axis_index("s") * NUM_CORES + lax.axis_index("c")
        col0 = wid * rows_per_w

        pltpu.sync_copy(idxt_hbm.at[:, pl.ds(col0, rows_per_w)], idx_v)

        lanes = [lax.broadcasted_iota(jnp.int32, (LANES,), 0) + h * LANES
                 for h in range(EMBED_DIM // LANES)]

        def offsets(g):
            return idx_v.at[g // 2, pl.ds((g % 2) * half, half)]

        def out_slice(g):
            return outt_hbm.at[g // 2, :,
                               pl.ds(col0 + (g % 2) * half, half)]

        def start_gather(g, rows, sem):
            pltpu.async_copy(table_hbm.at[offsets(g)], rows, sem)

        def finish(g, rows, tb, gsem, osem, p):
            pltpu.make_async_copy(table_hbm.at[offsets(g)], rows,
                                  gsem).wait()

            @pl.when(p > 0)
            def _():
                pltpu.make_async_copy(tb.at[:, pl.ds(0, half)],
                                      out_slice(g - 2), osem).wait()

            def scale_t(i, carry):
                icol = jnp.full((LANES,), 0, jnp.int32) + i
                for h in range(EMBED_DIM // LANES):
                    v = rows[i, pl.ds(h * LANES, LANES)] * SCALE
                    plsc.store_scatter(tb, [lanes[h], icol], v)
                return carry

            lax.fori_loop(0, half, scale_t, 0, unroll=8)
            pltpu.async_copy(tb.at[:, pl.ds(0, half)], out_slice(g), osem)

        start_gather(0, rows0, gs0)

        def pair_body(p, carry):
            g0 = 2 * p
            start_gather(g0 + 1, rows1, gs1)
            finish(g0, rows0, tb0, gs0, os0, p)

            @pl.when(g0 + 2 < n_chunks)
            def _():
                start_gather(g0 + 2, rows0, gs0)

            finish(g0 + 1, rows1, tb1, gs1, os1, p)
            return carry

        lax.fori_loop(0, n_chunks // 2, pair_body, 0)
        pltpu.make_async_copy(tb0.at[:, pl.ds(0, half)],
                              out_slice(n_chunks - 2), os0).wait()
        pltpu.make_async_copy(tb1.at[:, pl.ds(0, half)],
                              out_slice(n_chunks - 1), os1).wait()

    return sc_lookup


def kernel(input, table):
    n_rows, n_cols = input.shape
    idxt = input.T.astype(jnp.int32)
    tpad = jnp.pad(table, ((0, 0), (0, TABLE_PITCH - EMBED_DIM)))
    outt = _make_sc_lookup(n_rows, n_cols)(tpad, idxt)
    return jnp.transpose(outt, (2, 0, 1))
